# baseline (speedup 1.0000x reference)
import functools
import math

import jax
import jax.numpy as jnp
from jax import lax
from jax.experimental import pallas as pl
from jax.experimental.pallas import tpu as pltpu
from jax.experimental.pallas import tpu_sc as plsc

_BN_EPS = 1e-5
_NC = 2
_NS = 16
_NW = _NC * _NS
_CW = 64
_G = 64


@functools.lru_cache(maxsize=None)
def _make_sc_agg(n_acc, d, chunks_per_w):
    rows = n_acc // _NS
    p = 4
    g = 2
    n_stage = 4
    stage_w = chunks_per_w // n_stage
    mesh = plsc.VectorSubcoreMesh(core_axis_name="c", subcore_axis_name="s")

    @functools.partial(
        pl.kernel,
        mesh=mesh,
        out_type=jax.ShapeDtypeStruct((_NC, n_acc, d), jnp.float32),
        scratch_types=[
            pltpu.VMEM((stage_w, _CW), jnp.int32),
            pltpu.VMEM((stage_w, _CW), jnp.int32),
            pltpu.VMEM((p, _CW, d), jnp.float32),
            pltpu.VMEM_SHARED((n_acc, d), jnp.float32),
            pltpu.SemaphoreType.DMA((p,)),
            pltpu.SemaphoreType.DMA((p,)),
        ],
    )
    def agg(h_hbm, src_hbm, dst_hbm, zero_hbm, out_hbm,
            src_v, dst_v, rows_v, acc_sh, gsem, ssem):
        c = lax.axis_index("c")
        s = lax.axis_index("s")
        wid = s * _NC + c
        pltpu.sync_copy(zero_hbm.at[pl.ds(s * rows, rows)],
                        acc_sh.at[pl.ds(s * rows, rows)])
        plsc.subcore_barrier()

        for si in range(n_stage):
            base = wid * chunks_per_w + si * stage_w
            pltpu.sync_copy(src_hbm.at[pl.ds(base, stage_w)], src_v)
            pltpu.sync_copy(dst_hbm.at[pl.ds(base, stage_w)], dst_v)

            def prime(j, carry):
                pltpu.async_copy(h_hbm.at[src_v.at[j]], rows_v.at[j],
                                 gsem.at[j])
                return carry

            lax.fori_loop(0, g, prime, 0)

            def step(j, carry):
                slot = lax.rem(j, p)
                pltpu.make_async_copy(h_hbm.at[src_v.at[j]], rows_v.at[slot],
                                      gsem.at[slot]).wait()
                pltpu.async_copy(rows_v.at[slot], acc_sh.at[dst_v.at[j]],
                                 ssem.at[slot], add=True)

                jn = j + g

                @pl.when(jn < stage_w)
                def _():
                    sn = lax.rem(jn, p)

                    @pl.when(jn >= p)
                    def _():
                        pltpu.make_async_copy(rows_v.at[sn],
                                              acc_sh.at[dst_v.at[j]],
                                              ssem.at[sn]).wait()

                    pltpu.async_copy(h_hbm.at[src_v.at[jn]],
                                     rows_v.at[sn], gsem.at[sn])

                return carry

            lax.fori_loop(0, stage_w, step, 0)

            def drain(k, carry):
                m = stage_w - p + k
                pltpu.make_async_copy(rows_v.at[lax.rem(m, p)],
                                      acc_sh.at[dst_v.at[m]],
                                      ssem.at[lax.rem(m, p)]).wait()
                return carry

            lax.fori_loop(0, p, drain, 0)
        plsc.subcore_barrier()
        pltpu.sync_copy(acc_sh.at[pl.ds(s * rows, rows)],
                        out_hbm.at[c].at[pl.ds(s * rows, rows)])

    return agg


def _mlp_body(h_ref, a0_ref, a1_ref, eps_ref, w1_ref, b1_ref, w2_ref, b2_ref,
              o_ref):
    z = a0_ref[0] + a1_ref[0] + (1.0 + eps_ref[0]) * h_ref[...]
    z = jnp.maximum(z @ w1_ref[...] + b1_ref[...], 0.0)
    z = jnp.maximum(z @ w2_ref[...] + b2_ref[...], 0.0)
    o_ref[...] = z


def _mlp_pool_body(h_ref, a0_ref, a1_ref, eps_ref, w1_ref, b1_ref, w2_ref,
                   b2_ref, bat_ref, p1w_ref, p1b_ref, p2w_ref, p2b_ref,
                   o_ref, pool_ref):
    i = pl.program_id(0)
    nblk = pl.num_programs(0)
    z = a0_ref[0] + a1_ref[0] + (1.0 + eps_ref[0]) * h_ref[...]
    z = jnp.maximum(z @ w1_ref[...] + b1_ref[...], 0.0)
    z = jnp.maximum(z @ w2_ref[...] + b2_ref[...], 0.0)

    @pl.when(i == 0)
    def _():
        pool_ref[...] = jnp.zeros_like(pool_ref)

    b = bat_ref[0, 0]
    onehot = (b[:, None] == lax.broadcasted_iota(jnp.int32, (b.shape[0], _G),
                                                 1)).astype(jnp.float32)
    pool_ref[...] += lax.dot_general(onehot, z, (((0,), (0,)), ((), ())))

    @pl.when(i == nblk - 1)
    def _():
        p = pool_ref[...]
        t = jnp.maximum(p @ p1w_ref[...] + p1b_ref[...], 0.0)
        o_ref[...] = t @ p2w_ref[...] + p2b_ref[...]


def _tc_mlp(h, agg, eps, w1, b1, w2, b2, bm):
    n, d = h.shape
    h2 = w1.shape[1]
    nblk = n // bm
    return pl.pallas_call(
        _mlp_body,
        grid=(nblk,),
        in_specs=[
            pl.BlockSpec((bm, d), lambda i: (i, 0)),
            pl.BlockSpec((1, bm, d), lambda i: (0, i, 0)),
            pl.BlockSpec((1, bm, d), lambda i: (1, i, 0)),
            pl.BlockSpec(memory_space=pltpu.SMEM),
            pl.BlockSpec((d, h2), lambda i: (0, 0)),
            pl.BlockSpec((1, h2), lambda i: (0, 0)),
            pl.BlockSpec((h2, d), lambda i: (0, 0)),
            pl.BlockSpec((1, d), lambda i: (0, 0)),
        ],
        out_specs=pl.BlockSpec((bm, d), lambda i: (i, 0)),
        out_shape=jax.ShapeDtypeStruct((n, d), jnp.float32),
    )(h, agg, agg, eps, w1, b1, w2, b2)


def _tc_mlp_pool(h, agg, eps, w1, b1, w2, b2, bat3d, p1w, p1b, p2w, p2b, bm):
    n, d = h.shape
    h2 = w1.shape[1]
    out = p2w.shape[1]
    nblk = n // bm
    return pl.pallas_call(
        _mlp_pool_body,
        grid=(nblk,),
        in_specs=[
            pl.BlockSpec((bm, d), lambda i: (i, 0)),
            pl.BlockSpec((1, bm, d), lambda i: (0, i, 0)),
            pl.BlockSpec((1, bm, d), lambda i: (1, i, 0)),
            pl.BlockSpec(memory_space=pltpu.SMEM),
            pl.BlockSpec((d, h2), lambda i: (0, 0)),
            pl.BlockSpec((1, h2), lambda i: (0, 0)),
            pl.BlockSpec((h2, d), lambda i: (0, 0)),
            pl.BlockSpec((1, d), lambda i: (0, 0)),
            pl.BlockSpec((1, 1, bm), lambda i: (i, 0, 0)),
            pl.BlockSpec((d, d), lambda i: (0, 0)),
            pl.BlockSpec((1, d), lambda i: (0, 0)),
            pl.BlockSpec((d, out), lambda i: (0, 0)),
            pl.BlockSpec((1, out), lambda i: (0, 0)),
        ],
        out_specs=pl.BlockSpec((_G, out), lambda i: (0, 0)),
        out_shape=jax.ShapeDtypeStruct((_G, out), jnp.float32),
        scratch_shapes=[pltpu.VMEM((_G, d), jnp.float32)],
    )(h, agg, agg, eps, w1, b1, w2, b2, bat3d, p1w, p1b, p2w, p2b)


def kernel(x, edge_index, batch, params):
    n, d = x.shape
    e = edge_index.shape[1]
    c = math.sqrt(1.0 + _BN_EPS)
    bm = 2000

    e_pad = -(-e // (_NW * _CW * 8)) * (_NW * _CW * 8)
    chunks_per_w = e_pad // (_NW * _CW)
    npad = e_pad - e
    n_acc = -(-n // 128) * 128
    src = edge_index[0]
    dst = edge_index[1]
    if npad:
        fill = jnp.arange(npad, dtype=jnp.int32)
        src = jnp.concatenate([src, (fill * 7) % n])
        dst = jnp.concatenate([dst, n + (fill % (n_acc - n))])
    src2d = src.reshape(_NW * chunks_per_w, _CW)
    dst2d = dst.reshape(_NW * chunks_per_w, _CW)
    zeros = jnp.zeros((n_acc, d), jnp.float32)
    bat3d = batch.reshape(n // bm, 1, bm)

    sc_agg = _make_sc_agg(n_acc, d, chunks_per_w)

    h = x
    nl = len(params["layers"])
    for li, lp in enumerate(params["layers"]):
        agg = sc_agg(h, src2d, dst2d, zeros)
        s1 = lp["g1"] / c
        w1 = lp["W1"] * s1[None, :]
        b1 = (lp["b1"] * s1 + lp["be1"]).reshape(1, -1)
        s2 = lp["g2"] / c
        w2 = lp["W2"] * s2[None, :]
        b2 = (lp["b2"] * s2 + lp["be2"]).reshape(1, -1)
        eps = lp["eps"].reshape(1)
        if li < nl - 1:
            h = _tc_mlp(h, agg, eps, w1, b1, w2, b2, bm)
        else:
            sp = params["bn1_g"] / c
            p1w = params["lin1_W"] * sp[None, :]
            p1b = (params["lin1_b"] * sp + params["bn1_b"]).reshape(1, -1)
            p2w = params["lin2_W"]
            p2b = params["lin2_b"].reshape(1, -1)
            return _tc_mlp_pool(h, agg, eps, w1, b1, w2, b2, bat3d,
                                p1w, p1b, p2w, p2b, bm)


# speedup vs baseline: 1.1096x; 1.1096x over previous
import functools
import math

import jax
import jax.numpy as jnp
from jax import lax
from jax.experimental import pallas as pl
from jax.experimental.pallas import tpu as pltpu
from jax.experimental.pallas import tpu_sc as plsc

_BN_EPS = 1e-5
_NC = 2
_NS = 16
_NW = _NC * _NS
_CW = 64
_G = 64


@functools.lru_cache(maxsize=None)
def _make_sc_agg(n_acc, d, chunks_per_w):
    rows = n_acc // _NS
    p = 4
    g = 3
    n_stage = 4
    stage_w = chunks_per_w // n_stage
    mesh = plsc.VectorSubcoreMesh(core_axis_name="c", subcore_axis_name="s")

    @functools.partial(
        pl.kernel,
        mesh=mesh,
        out_type=jax.ShapeDtypeStruct((_NC, n_acc, d), jnp.float32),
        scratch_types=[
            pltpu.VMEM((stage_w, _CW), jnp.int32),
            pltpu.VMEM((stage_w, _CW), jnp.int32),
            pltpu.VMEM((p, _CW, d), jnp.float32),
            pltpu.VMEM_SHARED((n_acc, d), jnp.float32),
            pltpu.SemaphoreType.DMA((p,)),
            pltpu.SemaphoreType.DMA((p,)),
        ],
    )
    def agg(h_hbm, src_hbm, dst_hbm, zero_hbm, out_hbm,
            src_v, dst_v, rows_v, acc_sh, gsem, ssem):
        c = lax.axis_index("c")
        s = lax.axis_index("s")
        wid = s * _NC + c
        pltpu.sync_copy(zero_hbm.at[pl.ds(s * rows, rows)],
                        acc_sh.at[pl.ds(s * rows, rows)])
        plsc.subcore_barrier()

        for si in range(n_stage):
            base = wid * chunks_per_w + si * stage_w
            pltpu.sync_copy(src_hbm.at[pl.ds(base, stage_w)], src_v)
            pltpu.sync_copy(dst_hbm.at[pl.ds(base, stage_w)], dst_v)

            def prime(j, carry):
                pltpu.async_copy(h_hbm.at[src_v.at[j]], rows_v.at[j],
                                 gsem.at[j])
                return carry

            lax.fori_loop(0, g, prime, 0)

            def step(j, carry):
                slot = lax.rem(j, p)
                pltpu.make_async_copy(h_hbm.at[src_v.at[j]], rows_v.at[slot],
                                      gsem.at[slot]).wait()
                pltpu.async_copy(rows_v.at[slot], acc_sh.at[dst_v.at[j]],
                                 ssem.at[slot], add=True)

                jn = j + g

                @pl.when(jn < stage_w)
                def _():
                    sn = lax.rem(jn, p)

                    @pl.when(jn >= p)
                    def _():
                        pltpu.make_async_copy(rows_v.at[sn],
                                              acc_sh.at[dst_v.at[j]],
                                              ssem.at[sn]).wait()

                    pltpu.async_copy(h_hbm.at[src_v.at[jn]],
                                     rows_v.at[sn], gsem.at[sn])

                return carry

            lax.fori_loop(0, stage_w, step, 0)

            def drain(k, carry):
                m = stage_w - p + k
                pltpu.make_async_copy(rows_v.at[lax.rem(m, p)],
                                      acc_sh.at[dst_v.at[m]],
                                      ssem.at[lax.rem(m, p)]).wait()
                return carry

            lax.fori_loop(0, p, drain, 0)
        plsc.subcore_barrier()
        pltpu.sync_copy(acc_sh.at[pl.ds(s * rows, rows)],
                        out_hbm.at[c].at[pl.ds(s * rows, rows)])

    return agg


def _mlp_body(h_ref, a0_ref, a1_ref, eps_ref, w1_ref, b1_ref, w2_ref, b2_ref,
              o_ref):
    z = a0_ref[0] + a1_ref[0] + (1.0 + eps_ref[0]) * h_ref[...]
    z = jnp.maximum(z @ w1_ref[...] + b1_ref[...], 0.0)
    z = jnp.maximum(z @ w2_ref[...] + b2_ref[...], 0.0)
    o_ref[...] = z


def _mlp_pool_body(h_ref, a0_ref, a1_ref, eps_ref, w1_ref, b1_ref, w2_ref,
                   b2_ref, bat_ref, p1w_ref, p1b_ref, p2w_ref, p2b_ref,
                   o_ref, pool_ref):
    i = pl.program_id(0)
    nblk = pl.num_programs(0)
    z = a0_ref[0] + a1_ref[0] + (1.0 + eps_ref[0]) * h_ref[...]
    z = jnp.maximum(z @ w1_ref[...] + b1_ref[...], 0.0)
    z = jnp.maximum(z @ w2_ref[...] + b2_ref[...], 0.0)

    @pl.when(i == 0)
    def _():
        pool_ref[...] = jnp.zeros_like(pool_ref)

    b = bat_ref[0, 0]
    onehot = (b[:, None] == lax.broadcasted_iota(jnp.int32, (b.shape[0], _G),
                                                 1)).astype(jnp.float32)
    pool_ref[...] += lax.dot_general(onehot, z, (((0,), (0,)), ((), ())))

    @pl.when(i == nblk - 1)
    def _():
        p = pool_ref[...]
        t = jnp.maximum(p @ p1w_ref[...] + p1b_ref[...], 0.0)
        o_ref[...] = t @ p2w_ref[...] + p2b_ref[...]


def _tc_mlp(h, agg, eps, w1, b1, w2, b2, bm):
    n, d = h.shape
    h2 = w1.shape[1]
    nblk = n // bm
    return pl.pallas_call(
        _mlp_body,
        grid=(nblk,),
        in_specs=[
            pl.BlockSpec((bm, d), lambda i: (i, 0)),
            pl.BlockSpec((1, bm, d), lambda i: (0, i, 0)),
            pl.BlockSpec((1, bm, d), lambda i: (1, i, 0)),
            pl.BlockSpec(memory_space=pltpu.SMEM),
            pl.BlockSpec((d, h2), lambda i: (0, 0)),
            pl.BlockSpec((1, h2), lambda i: (0, 0)),
            pl.BlockSpec((h2, d), lambda i: (0, 0)),
            pl.BlockSpec((1, d), lambda i: (0, 0)),
        ],
        out_specs=pl.BlockSpec((bm, d), lambda i: (i, 0)),
        out_shape=jax.ShapeDtypeStruct((n, d), jnp.float32),
    )(h, agg, agg, eps, w1, b1, w2, b2)


def _tc_mlp_pool(h, agg, eps, w1, b1, w2, b2, bat3d, p1w, p1b, p2w, p2b, bm):
    n, d = h.shape
    h2 = w1.shape[1]
    out = p2w.shape[1]
    nblk = n // bm
    return pl.pallas_call(
        _mlp_pool_body,
        grid=(nblk,),
        in_specs=[
            pl.BlockSpec((bm, d), lambda i: (i, 0)),
            pl.BlockSpec((1, bm, d), lambda i: (0, i, 0)),
            pl.BlockSpec((1, bm, d), lambda i: (1, i, 0)),
            pl.BlockSpec(memory_space=pltpu.SMEM),
            pl.BlockSpec((d, h2), lambda i: (0, 0)),
            pl.BlockSpec((1, h2), lambda i: (0, 0)),
            pl.BlockSpec((h2, d), lambda i: (0, 0)),
            pl.BlockSpec((1, d), lambda i: (0, 0)),
            pl.BlockSpec((1, 1, bm), lambda i: (i, 0, 0)),
            pl.BlockSpec((d, d), lambda i: (0, 0)),
            pl.BlockSpec((1, d), lambda i: (0, 0)),
            pl.BlockSpec((d, out), lambda i: (0, 0)),
            pl.BlockSpec((1, out), lambda i: (0, 0)),
        ],
        out_specs=pl.BlockSpec((_G, out), lambda i: (0, 0)),
        out_shape=jax.ShapeDtypeStruct((_G, out), jnp.float32),
        scratch_shapes=[pltpu.VMEM((_G, d), jnp.float32)],
    )(h, agg, agg, eps, w1, b1, w2, b2, bat3d, p1w, p1b, p2w, p2b)


def kernel(x, edge_index, batch, params):
    n, d = x.shape
    e = edge_index.shape[1]
    c = math.sqrt(1.0 + _BN_EPS)
    bm = 2000

    e_pad = -(-e // (_NW * _CW * 8)) * (_NW * _CW * 8)
    chunks_per_w = e_pad // (_NW * _CW)
    npad = e_pad - e
    n_acc = -(-n // 128) * 128
    src = edge_index[0]
    dst = edge_index[1]
    if npad:
        fill = jnp.arange(npad, dtype=jnp.int32)
        src = jnp.concatenate([src, (fill * 7) % n])
        dst = jnp.concatenate([dst, n + (fill % (n_acc - n))])
    src2d = src.reshape(_NW * chunks_per_w, _CW)
    dst2d = dst.reshape(_NW * chunks_per_w, _CW)
    zeros = jnp.zeros((n_acc, d), jnp.float32)
    bat3d = batch.reshape(n // bm, 1, bm)

    sc_agg = _make_sc_agg(n_acc, d, chunks_per_w)

    h = x
    nl = len(params["layers"])
    for li, lp in enumerate(params["layers"]):
        agg = sc_agg(h, src2d, dst2d, zeros)
        s1 = lp["g1"] / c
        w1 = lp["W1"] * s1[None, :]
        b1 = (lp["b1"] * s1 + lp["be1"]).reshape(1, -1)
        s2 = lp["g2"] / c
        w2 = lp["W2"] * s2[None, :]
        b2 = (lp["b2"] * s2 + lp["be2"]).reshape(1, -1)
        eps = lp["eps"].reshape(1)
        if li < nl - 1:
            h = _tc_mlp(h, agg, eps, w1, b1, w2, b2, bm)
        else:
            sp = params["bn1_g"] / c
            p1w = params["lin1_W"] * sp[None, :]
            p1b = (params["lin1_b"] * sp + params["bn1_b"]).reshape(1, -1)
            p2w = params["lin2_W"]
            p2b = params["lin2_b"].reshape(1, -1)
            return _tc_mlp_pool(h, agg, eps, w1, b1, w2, b2, bat3d,
                                p1w, p1b, p2w, p2b, bm)


# speedup vs baseline: 1.1344x; 1.0224x over previous
import functools
import math

import jax
import jax.numpy as jnp
from jax import lax
from jax.experimental import pallas as pl
from jax.experimental.pallas import tpu as pltpu
from jax.experimental.pallas import tpu_sc as plsc

_BN_EPS = 1e-5
_NC = 2
_NS = 16
_NW = _NC * _NS
_CW = 128
_G = 64


@functools.lru_cache(maxsize=None)
def _make_sc_agg(n_acc, d, chunks_per_w):
    rows = n_acc // _NS
    p = 2
    n_stage = 2
    stage_w = chunks_per_w // n_stage
    mesh = plsc.VectorSubcoreMesh(core_axis_name="c", subcore_axis_name="s")

    @functools.partial(
        pl.kernel,
        mesh=mesh,
        out_type=jax.ShapeDtypeStruct((_NC, n_acc, d), jnp.float32),
        scratch_types=[
            pltpu.VMEM((stage_w, _CW), jnp.int32),
            pltpu.VMEM((stage_w, _CW), jnp.int32),
            pltpu.VMEM((p, _CW, d), jnp.float32),
            pltpu.VMEM_SHARED((n_acc, d), jnp.float32),
            pltpu.SemaphoreType.DMA((p,)),
        ],
    )
    def agg(h_hbm, src_hbm, dst_hbm, zero_hbm, out_hbm,
            src_v, dst_v, rows_v, acc_sh, gsem):
        c = lax.axis_index("c")
        s = lax.axis_index("s")
        wid = s * _NC + c
        pltpu.sync_copy(zero_hbm.at[pl.ds(s * rows, rows)],
                        acc_sh.at[pl.ds(s * rows, rows)])
        plsc.subcore_barrier()

        for si in range(n_stage):
            base = wid * chunks_per_w + si * stage_w
            pltpu.sync_copy(src_hbm.at[pl.ds(base, stage_w)], src_v)
            pltpu.sync_copy(dst_hbm.at[pl.ds(base, stage_w)], dst_v)

            def prime(j, carry):
                pltpu.async_copy(h_hbm.at[src_v.at[j]], rows_v.at[j],
                                 gsem.at[j])
                return carry

            lax.fori_loop(0, p, prime, 0)

            def step(j, carry):
                slot = lax.rem(j, p)
                pltpu.make_async_copy(h_hbm.at[src_v.at[j]], rows_v.at[slot],
                                      gsem.at[slot]).wait()
                pltpu.sync_copy(rows_v.at[slot], acc_sh.at[dst_v.at[j]],
                                add=True)

                @pl.when(j + p < stage_w)
                def _():
                    pltpu.async_copy(h_hbm.at[src_v.at[j + p]],
                                     rows_v.at[slot], gsem.at[slot])

                return carry

            lax.fori_loop(0, stage_w, step, 0)
        plsc.subcore_barrier()
        pltpu.sync_copy(acc_sh.at[pl.ds(s * rows, rows)],
                        out_hbm.at[c].at[pl.ds(s * rows, rows)])

    return agg


def _mlp_body(h_ref, a0_ref, a1_ref, eps_ref, w1_ref, b1_ref, w2_ref, b2_ref,
              o_ref):
    z = a0_ref[0] + a1_ref[0] + (1.0 + eps_ref[0]) * h_ref[...]
    z = jnp.maximum(z @ w1_ref[...] + b1_ref[...], 0.0)
    z = jnp.maximum(z @ w2_ref[...] + b2_ref[...], 0.0)
    o_ref[...] = z


def _mlp_pool_body(h_ref, a0_ref, a1_ref, eps_ref, w1_ref, b1_ref, w2_ref,
                   b2_ref, bat_ref, p1w_ref, p1b_ref, p2w_ref, p2b_ref,
                   o_ref, pool_ref):
    i = pl.program_id(0)
    nblk = pl.num_programs(0)
    z = a0_ref[0] + a1_ref[0] + (1.0 + eps_ref[0]) * h_ref[...]
    z = jnp.maximum(z @ w1_ref[...] + b1_ref[...], 0.0)
    z = jnp.maximum(z @ w2_ref[...] + b2_ref[...], 0.0)

    @pl.when(i == 0)
    def _():
        pool_ref[...] = jnp.zeros_like(pool_ref)

    b = bat_ref[0, 0]
    onehot = (b[:, None] == lax.broadcasted_iota(jnp.int32, (b.shape[0], _G),
                                                 1)).astype(jnp.float32)
    pool_ref[...] += lax.dot_general(onehot, z, (((0,), (0,)), ((), ())))

    @pl.when(i == nblk - 1)
    def _():
        p = pool_ref[...]
        t = jnp.maximum(p @ p1w_ref[...] + p1b_ref[...], 0.0)
        o_ref[...] = t @ p2w_ref[...] + p2b_ref[...]


def _tc_mlp(h, agg, eps, w1, b1, w2, b2, bm):
    n, d = h.shape
    h2 = w1.shape[1]
    nblk = n // bm
    return pl.pallas_call(
        _mlp_body,
        grid=(nblk,),
        in_specs=[
            pl.BlockSpec((bm, d), lambda i: (i, 0)),
            pl.BlockSpec((1, bm, d), lambda i: (0, i, 0)),
            pl.BlockSpec((1, bm, d), lambda i: (1, i, 0)),
            pl.BlockSpec(memory_space=pltpu.SMEM),
            pl.BlockSpec((d, h2), lambda i: (0, 0)),
            pl.BlockSpec((1, h2), lambda i: (0, 0)),
            pl.BlockSpec((h2, d), lambda i: (0, 0)),
            pl.BlockSpec((1, d), lambda i: (0, 0)),
        ],
        out_specs=pl.BlockSpec((bm, d), lambda i: (i, 0)),
        out_shape=jax.ShapeDtypeStruct((n, d), jnp.float32),
    )(h, agg, agg, eps, w1, b1, w2, b2)


def _tc_mlp_pool(h, agg, eps, w1, b1, w2, b2, bat3d, p1w, p1b, p2w, p2b, bm):
    n, d = h.shape
    h2 = w1.shape[1]
    out = p2w.shape[1]
    nblk = n // bm
    return pl.pallas_call(
        _mlp_pool_body,
        grid=(nblk,),
        in_specs=[
            pl.BlockSpec((bm, d), lambda i: (i, 0)),
            pl.BlockSpec((1, bm, d), lambda i: (0, i, 0)),
            pl.BlockSpec((1, bm, d), lambda i: (1, i, 0)),
            pl.BlockSpec(memory_space=pltpu.SMEM),
            pl.BlockSpec((d, h2), lambda i: (0, 0)),
            pl.BlockSpec((1, h2), lambda i: (0, 0)),
            pl.BlockSpec((h2, d), lambda i: (0, 0)),
            pl.BlockSpec((1, d), lambda i: (0, 0)),
            pl.BlockSpec((1, 1, bm), lambda i: (i, 0, 0)),
            pl.BlockSpec((d, d), lambda i: (0, 0)),
            pl.BlockSpec((1, d), lambda i: (0, 0)),
            pl.BlockSpec((d, out), lambda i: (0, 0)),
            pl.BlockSpec((1, out), lambda i: (0, 0)),
        ],
        out_specs=pl.BlockSpec((_G, out), lambda i: (0, 0)),
        out_shape=jax.ShapeDtypeStruct((_G, out), jnp.float32),
        scratch_shapes=[pltpu.VMEM((_G, d), jnp.float32)],
    )(h, agg, agg, eps, w1, b1, w2, b2, bat3d, p1w, p1b, p2w, p2b)


def kernel(x, edge_index, batch, params):
    n, d = x.shape
    e = edge_index.shape[1]
    c = math.sqrt(1.0 + _BN_EPS)
    bm = 2000

    e_pad = -(-e // (_NW * _CW * 8)) * (_NW * _CW * 8)
    chunks_per_w = e_pad // (_NW * _CW)
    npad = e_pad - e
    n_acc = -(-n // 128) * 128
    src = edge_index[0]
    dst = edge_index[1]
    if npad:
        fill = jnp.arange(npad, dtype=jnp.int32)
        src = jnp.concatenate([src, (fill * 7) % n])
        dst = jnp.concatenate([dst, n + (fill % (n_acc - n))])
    src2d = src.reshape(_NW * chunks_per_w, _CW)
    dst2d = dst.reshape(_NW * chunks_per_w, _CW)
    zeros = jnp.zeros((n_acc, d), jnp.float32)
    bat3d = batch.reshape(n // bm, 1, bm)

    sc_agg = _make_sc_agg(n_acc, d, chunks_per_w)

    h = x
    nl = len(params["layers"])
    for li, lp in enumerate(params["layers"]):
        agg = sc_agg(h, src2d, dst2d, zeros)
        s1 = lp["g1"] / c
        w1 = lp["W1"] * s1[None, :]
        b1 = (lp["b1"] * s1 + lp["be1"]).reshape(1, -1)
        s2 = lp["g2"] / c
        w2 = lp["W2"] * s2[None, :]
        b2 = (lp["b2"] * s2 + lp["be2"]).reshape(1, -1)
        eps = lp["eps"].reshape(1)
        if li < nl - 1:
            h = _tc_mlp(h, agg, eps, w1, b1, w2, b2, bm)
        else:
            sp = params["bn1_g"] / c
            p1w = params["lin1_W"] * sp[None, :]
            p1b = (params["lin1_b"] * sp + params["bn1_b"]).reshape(1, -1)
            p2w = params["lin2_W"]
            p2b = params["lin2_b"].reshape(1, -1)
            return _tc_mlp_pool(h, agg, eps, w1, b1, w2, b2, bat3d,
                                p1w, p1b, p2w, p2b, bm)


# speedup vs baseline: 1.1849x; 1.0445x over previous
import functools
import math

import jax
import jax.numpy as jnp
from jax import lax
from jax.experimental import pallas as pl
from jax.experimental.pallas import tpu as pltpu
from jax.experimental.pallas import tpu_sc as plsc

_BN_EPS = 1e-5
_NC = 2
_NS = 16
_NW = _NC * _NS
_CW = 64
_G = 64


@functools.lru_cache(maxsize=None)
def _make_sc_agg(n_acc, d, chunks_per_w):
    rows = n_acc // _NS
    p = 4
    n_stage = 4
    stage_w = chunks_per_w // n_stage
    mesh = plsc.VectorSubcoreMesh(core_axis_name="c", subcore_axis_name="s")

    @functools.partial(
        pl.kernel,
        mesh=mesh,
        out_type=jax.ShapeDtypeStruct((_NC, n_acc, d), jnp.float32),
        scratch_types=[
            pltpu.VMEM((stage_w, _CW), jnp.int32),
            pltpu.VMEM((stage_w, _CW), jnp.int32),
            pltpu.VMEM((p, _CW, d), jnp.float32),
            pltpu.VMEM_SHARED((n_acc, d), jnp.float32),
            pltpu.SemaphoreType.DMA((p,)),
        ],
    )
    def agg(h_hbm, src_hbm, dst_hbm, zero_hbm, out_hbm,
            src_v, dst_v, rows_v, acc_sh, gsem):
        c = lax.axis_index("c")
        s = lax.axis_index("s")
        wid = s * _NC + c
        pltpu.sync_copy(zero_hbm.at[pl.ds(s * rows, rows)],
                        acc_sh.at[pl.ds(s * rows, rows)])
        plsc.subcore_barrier()

        for si in range(n_stage):
            base = wid * chunks_per_w + si * stage_w
            pltpu.sync_copy(src_hbm.at[pl.ds(base, stage_w)], src_v)
            pltpu.sync_copy(dst_hbm.at[pl.ds(base, stage_w)], dst_v)

            def prime(j, carry):
                pltpu.async_copy(h_hbm.at[src_v.at[j]], rows_v.at[j],
                                 gsem.at[j])
                return carry

            lax.fori_loop(0, p, prime, 0)

            def step(j, carry):
                slot = lax.rem(j, p)
                pltpu.make_async_copy(h_hbm.at[src_v.at[j]], rows_v.at[slot],
                                      gsem.at[slot]).wait()
                pltpu.sync_copy(rows_v.at[slot], acc_sh.at[dst_v.at[j]],
                                add=True)

                @pl.when(j + p < stage_w)
                def _():
                    pltpu.async_copy(h_hbm.at[src_v.at[j + p]],
                                     rows_v.at[slot], gsem.at[slot])

                return carry

            lax.fori_loop(0, stage_w, step, 0)
        plsc.subcore_barrier()
        pltpu.sync_copy(acc_sh.at[pl.ds(s * rows, rows)],
                        out_hbm.at[c].at[pl.ds(s * rows, rows)])

    return agg


def _mlp_body(h_ref, a0_ref, a1_ref, eps_ref, w1_ref, b1_ref, w2_ref, b2_ref,
              o_ref):
    z = a0_ref[0] + a1_ref[0] + (1.0 + eps_ref[0]) * h_ref[...]
    z = jnp.maximum(z @ w1_ref[...] + b1_ref[...], 0.0)
    z = jnp.maximum(z @ w2_ref[...] + b2_ref[...], 0.0)
    o_ref[...] = z


def _mlp_pool_body(h_ref, a0_ref, a1_ref, eps_ref, w1_ref, b1_ref, w2_ref,
                   b2_ref, bat_ref, p1w_ref, p1b_ref, p2w_ref, p2b_ref,
                   o_ref, pool_ref):
    i = pl.program_id(0)
    nblk = pl.num_programs(0)
    z = a0_ref[0] + a1_ref[0] + (1.0 + eps_ref[0]) * h_ref[...]
    z = jnp.maximum(z @ w1_ref[...] + b1_ref[...], 0.0)
    z = jnp.maximum(z @ w2_ref[...] + b2_ref[...], 0.0)

    @pl.when(i == 0)
    def _():
        pool_ref[...] = jnp.zeros_like(pool_ref)

    b = bat_ref[0, 0]
    onehot = (b[:, None] == lax.broadcasted_iota(jnp.int32, (b.shape[0], _G),
                                                 1)).astype(jnp.float32)
    pool_ref[...] += lax.dot_general(onehot, z, (((0,), (0,)), ((), ())))

    @pl.when(i == nblk - 1)
    def _():
        p = pool_ref[...]
        t = jnp.maximum(p @ p1w_ref[...] + p1b_ref[...], 0.0)
        o_ref[...] = t @ p2w_ref[...] + p2b_ref[...]


def _tc_mlp(h, agg, eps, w1, b1, w2, b2, bm):
    n, d = h.shape
    h2 = w1.shape[1]
    nblk = n // bm
    return pl.pallas_call(
        _mlp_body,
        grid=(nblk,),
        in_specs=[
            pl.BlockSpec((bm, d), lambda i: (i, 0)),
            pl.BlockSpec((1, bm, d), lambda i: (0, i, 0)),
            pl.BlockSpec((1, bm, d), lambda i: (1, i, 0)),
            pl.BlockSpec(memory_space=pltpu.SMEM),
            pl.BlockSpec((d, h2), lambda i: (0, 0)),
            pl.BlockSpec((1, h2), lambda i: (0, 0)),
            pl.BlockSpec((h2, d), lambda i: (0, 0)),
            pl.BlockSpec((1, d), lambda i: (0, 0)),
        ],
        out_specs=pl.BlockSpec((bm, d), lambda i: (i, 0)),
        out_shape=jax.ShapeDtypeStruct((n, d), jnp.float32),
    )(h, agg, agg, eps, w1, b1, w2, b2)


def _tc_mlp_pool(h, agg, eps, w1, b1, w2, b2, bat3d, p1w, p1b, p2w, p2b, bm):
    n, d = h.shape
    h2 = w1.shape[1]
    out = p2w.shape[1]
    nblk = n // bm
    return pl.pallas_call(
        _mlp_pool_body,
        grid=(nblk,),
        in_specs=[
            pl.BlockSpec((bm, d), lambda i: (i, 0)),
            pl.BlockSpec((1, bm, d), lambda i: (0, i, 0)),
            pl.BlockSpec((1, bm, d), lambda i: (1, i, 0)),
            pl.BlockSpec(memory_space=pltpu.SMEM),
            pl.BlockSpec((d, h2), lambda i: (0, 0)),
            pl.BlockSpec((1, h2), lambda i: (0, 0)),
            pl.BlockSpec((h2, d), lambda i: (0, 0)),
            pl.BlockSpec((1, d), lambda i: (0, 0)),
            pl.BlockSpec((1, 1, bm), lambda i: (i, 0, 0)),
            pl.BlockSpec((d, d), lambda i: (0, 0)),
            pl.BlockSpec((1, d), lambda i: (0, 0)),
            pl.BlockSpec((d, out), lambda i: (0, 0)),
            pl.BlockSpec((1, out), lambda i: (0, 0)),
        ],
        out_specs=pl.BlockSpec((_G, out), lambda i: (0, 0)),
        out_shape=jax.ShapeDtypeStruct((_G, out), jnp.float32),
        scratch_shapes=[pltpu.VMEM((_G, d), jnp.float32)],
    )(h, agg, agg, eps, w1, b1, w2, b2, bat3d, p1w, p1b, p2w, p2b)


def kernel(x, edge_index, batch, params):
    n, d = x.shape
    e = edge_index.shape[1]
    c = math.sqrt(1.0 + _BN_EPS)
    bm = 2000

    e_pad = -(-e // (_NW * _CW * 8)) * (_NW * _CW * 8)
    chunks_per_w = e_pad // (_NW * _CW)
    npad = e_pad - e
    n_acc = -(-n // 128) * 128
    src = edge_index[0]
    dst = edge_index[1]
    if npad:
        fill = jnp.arange(npad, dtype=jnp.int32)
        src = jnp.concatenate([src, (fill * 7) % n])
        dst = jnp.concatenate([dst, n + (fill % (n_acc - n))])
    src2d = src.reshape(_NW * chunks_per_w, _CW)
    dst2d = dst.reshape(_NW * chunks_per_w, _CW)
    zeros = jnp.zeros((n_acc, d), jnp.float32)
    bat3d = batch.reshape(n // bm, 1, bm)

    sc_agg = _make_sc_agg(n_acc, d, chunks_per_w)

    h = x
    nl = len(params["layers"])
    for li, lp in enumerate(params["layers"]):
        agg = sc_agg(h, src2d, dst2d, zeros)
        s1 = lp["g1"] / c
        w1 = lp["W1"] * s1[None, :]
        b1 = (lp["b1"] * s1 + lp["be1"]).reshape(1, -1)
        s2 = lp["g2"] / c
        w2 = lp["W2"] * s2[None, :]
        b2 = (lp["b2"] * s2 + lp["be2"]).reshape(1, -1)
        eps = lp["eps"].reshape(1)
        if li < nl - 1:
            h = _tc_mlp(h, agg, eps, w1, b1, w2, b2, bm)
        else:
            sp = params["bn1_g"] / c
            p1w = params["lin1_W"] * sp[None, :]
            p1b = (params["lin1_b"] * sp + params["bn1_b"]).reshape(1, -1)
            p2w = params["lin2_W"]
            p2b = params["lin2_b"].reshape(1, -1)
            return _tc_mlp_pool(h, agg, eps, w1, b1, w2, b2, bat3d,
                                p1w, p1b, p2w, p2b, bm)
